# baseline (speedup 1.0000x reference)
import functools

import jax
import jax.numpy as jnp
from jax import lax
from jax.experimental import pallas as pl
from jax.experimental.pallas import tpu as pltpu
from jax.experimental.pallas import tpu_sc as plsc

MAXLEN = 200
EMBED = 64
LANES = 16
VPR = EMBED // LANES

NC, NS = 2, 16
NW = NC * NS

CHUNK = 128
NBUF = 8
LOOK = 4

_mesh = plsc.VectorSubcoreMesh(core_axis_name="c", subcore_axis_name="s")


def _emb_body(nch, x_hbm, wt_hbm, pos2_hbm, out_hbm, idx_v, pos_v, *rest):
    bufs = rest[:NBUF]
    gsems = rest[NBUF:2 * NBUF]
    ssems = rest[2 * NBUF:3 * NBUF]

    wid = lax.axis_index("s") * NC + lax.axis_index("c")
    rbase = wid * (nch * CHUNK)

    pltpu.sync_copy(x_hbm.at[pl.ds(wid * nch, nch)], idx_v)
    pltpu.sync_copy(pos2_hbm, pos_v)

    def issue_gather(c, b):
        pltpu.async_copy(wt_hbm.at[idx_v.at[c]], bufs[b], gsems[b])

    def wait_gather(c, b):
        pltpu.make_async_copy(wt_hbm.at[idx_v.at[c]], bufs[b], gsems[b]).wait()

    def issue_store(c, b):
        pltpu.async_copy(
            bufs[b], out_hbm.at[pl.ds(rbase + c * CHUNK, CHUNK)], ssems[b])

    def wait_store(c, b):
        pltpu.make_async_copy(
            bufs[b], out_hbm.at[pl.ds(rbase + c * CHUNK, CHUNK)], ssems[b]).wait()

    def add_pos(c, b):
        q = lax.rem(c * CHUNK, MAXLEN)
        buf = bufs[b]

        @plsc.parallel_loop(0, CHUNK, 1, unroll=4)
        def _(r):
            p = q + r
            for k in range(VPR):
                v = pos_v[p, pl.ds(k * LANES, LANES)]
                plsc.addupdate(buf.at[r, pl.ds(k * LANES, LANES)], v)

    def process(c, b):
        wait_gather(c, b)
        add_pos(c, b)
        issue_store(c, b)

    for b in range(LOOK):
        issue_gather(b, b)

    for c in range(LOOK):
        process(c, c % NBUF)
        issue_gather(c + LOOK, (c + LOOK) % NBUF)

    ngroups = (nch - 2 * LOOK) // NBUF

    def group(g, _):
        for j in range(NBUF):
            c = LOOK + g * NBUF + j
            b = (LOOK + j) % NBUF
            bf = (LOOK + j + LOOK) % NBUF
            wait_store(c - LOOK, bf)
            issue_gather(c + LOOK, bf)
            process(c, b)
        return _

    lax.fori_loop(0, ngroups, group, 0)

    for c in range(nch - LOOK, nch):
        process(c, c % NBUF)

    for c in range(nch - NBUF, nch):
        wait_store(c, c % NBUF)


def kernel(x, word_table, pos_table):
    batch, maxlen = x.shape
    rows = batch * maxlen
    assert maxlen == MAXLEN and word_table.shape[1] == EMBED
    rpw = rows // NW
    nch = rpw // CHUNK
    assert rpw % CHUNK == 0 and rpw % MAXLEN == 0
    assert (nch - 2 * LOOK) % NBUF == 0

    x2 = x.reshape(-1, CHUNK).astype(jnp.int32)
    pos2 = jnp.concatenate([pos_table, pos_table], axis=0)

    scratch = (
        [pltpu.VMEM((nch, CHUNK), jnp.int32),
         pltpu.VMEM((2 * MAXLEN, EMBED), jnp.float32)]
        + [pltpu.VMEM((CHUNK, EMBED), jnp.float32) for _ in range(NBUF)]
        + [pltpu.SemaphoreType.DMA for _ in range(2 * NBUF)]
    )
    out = pl.kernel(
        functools.partial(_emb_body, nch),
        out_type=jax.ShapeDtypeStruct((rows, EMBED), jnp.float32),
        mesh=_mesh,
        scratch_types=scratch,
        compiler_params=pltpu.CompilerParams(use_tc_tiling_on_sc=False),
    )(x2, word_table, pos2)
    return out.reshape(batch, maxlen, EMBED)


# speedup vs baseline: 1.6511x; 1.6511x over previous
import functools

import jax
import jax.numpy as jnp
from jax import lax
from jax.experimental import pallas as pl
from jax.experimental.pallas import tpu as pltpu
from jax.experimental.pallas import tpu_sc as plsc

MAXLEN = 200
EMBED = 64
WROW = 128
LANES = 16
VPR = EMBED // LANES

NC, NS = 2, 16
NW = NC * NS

CHUNK = 128
NBUF = 4
LOOK = 2

_mesh = plsc.VectorSubcoreMesh(core_axis_name="c", subcore_axis_name="s")


def _emb_body(nch, x_hbm, wt_hbm, pos2_hbm, out_hbm, idx_v, pos_v, *rest):
    bufs = rest[:NBUF]
    gsems = rest[NBUF:2 * NBUF]
    ssems = rest[2 * NBUF:3 * NBUF]

    wid = lax.axis_index("s") * NC + lax.axis_index("c")
    rbase = wid * (nch * CHUNK)

    pltpu.sync_copy(x_hbm.at[pl.ds(wid * nch, nch)], idx_v)
    pltpu.sync_copy(pos2_hbm, pos_v)

    def issue_gather(c, b):
        pltpu.async_copy(wt_hbm.at[idx_v.at[c]], bufs[b], gsems[b])

    def wait_gather(c, b):
        pltpu.make_async_copy(wt_hbm.at[idx_v.at[c]], bufs[b], gsems[b]).wait()

    def _store_refs(c, b):
        return (bufs[b],
                out_hbm.at[pl.ds(rbase + c * CHUNK, CHUNK), pl.ds(0, EMBED)])

    def issue_store(c, b):
        src, dst = _store_refs(c, b)
        pltpu.async_copy(src, dst, ssems[b])

    def wait_store(c, b):
        src, dst = _store_refs(c, b)
        pltpu.make_async_copy(src, dst, ssems[b]).wait()

    def add_pos(c, b):
        q = lax.rem(c * CHUNK, MAXLEN)
        buf = bufs[b]

        @plsc.parallel_loop(0, CHUNK, 1, unroll=4)
        def _(r):
            p = q + r
            for k in range(VPR):
                v = pos_v[p, pl.ds(k * LANES, LANES)]
                plsc.addupdate(buf.at[r, pl.ds(k * LANES, LANES)], v)

    def process(c, b):
        wait_gather(c, b)
        add_pos(c, b)
        issue_store(c, b)

    for b in range(LOOK):
        issue_gather(b, b)

    for c in range(LOOK):
        process(c, c % NBUF)
        issue_gather(c + LOOK, (c + LOOK) % NBUF)

    ngroups = (nch - 2 * LOOK) // NBUF

    def group(g, _):
        for j in range(NBUF):
            c = LOOK + g * NBUF + j
            b = (LOOK + j) % NBUF
            bf = (LOOK + j + LOOK) % NBUF
            wait_store(c - LOOK, bf)
            issue_gather(c + LOOK, bf)
            process(c, b)
        return _

    lax.fori_loop(0, ngroups, group, 0)

    for c in range(nch - LOOK, nch):
        process(c, c % NBUF)

    for c in range(nch - NBUF, nch):
        wait_store(c, c % NBUF)


def kernel(x, word_table, pos_table):
    batch, maxlen = x.shape
    rows = batch * maxlen
    assert maxlen == MAXLEN and word_table.shape[1] == EMBED
    rpw = rows // NW
    nch = rpw // CHUNK
    assert rpw % CHUNK == 0 and rpw % MAXLEN == 0
    assert (nch - 2 * LOOK) % NBUF == 0

    x2 = x.reshape(-1, CHUNK).astype(jnp.int32)
    wt2 = word_table
    pos2 = jnp.concatenate([pos_table, pos_table], axis=0)

    scratch = (
        [pltpu.VMEM((nch, CHUNK), jnp.int32),
         pltpu.VMEM((2 * MAXLEN, EMBED), jnp.float32)]
        + [pltpu.VMEM((CHUNK, EMBED), jnp.float32) for _ in range(NBUF)]
        + [pltpu.SemaphoreType.DMA for _ in range(2 * NBUF)]
    )
    out = pl.kernel(
        functools.partial(_emb_body, nch),
        out_type=jax.ShapeDtypeStruct((rows, WROW), jnp.float32),
        mesh=_mesh,
        scratch_types=scratch,
        compiler_params=pltpu.CompilerParams(use_tc_tiling_on_sc=False),
    )(x2, wt2, pos2)
    return out[:, :EMBED].reshape(batch, maxlen, EMBED)


# speedup vs baseline: 1.6523x; 1.0007x over previous
import functools

import jax
import jax.numpy as jnp
from jax import lax
from jax.experimental import pallas as pl
from jax.experimental.pallas import tpu as pltpu
from jax.experimental.pallas import tpu_sc as plsc

MAXLEN = 200
EMBED = 64
WROW = 128
LANES = 16
VPR = EMBED // LANES

NC, NS = 2, 16
NW = NC * NS

CHUNK = 128
NBUF = 4
LOOK = 2

_mesh = plsc.VectorSubcoreMesh(core_axis_name="c", subcore_axis_name="s")

VB = 8192


def _pack_body(in_ref, out_ref):
    tt = in_ref[...].T.reshape(VB // 2, 2, EMBED)
    out_ref[...] = jnp.concatenate([tt[:, 0, :], tt[:, 1, :]], axis=1)


def _tc_pack(wt_t):
    v = wt_t.shape[1]
    grid = -(-v // VB)
    out = pl.pallas_call(
        _pack_body,
        grid=(grid,),
        in_specs=[pl.BlockSpec((EMBED, VB), lambda g: (0, g))],
        out_specs=pl.BlockSpec((VB // 2, 128), lambda g: (g, 0)),
        out_shape=jax.ShapeDtypeStruct((v // 2, 128), jnp.float32),
    )(wt_t)
    return out.reshape(v, EMBED)


def _emb_body(nch, x_hbm, wt_hbm, pos2_hbm, out_hbm, idx_v, pos_v, *rest):
    bufs = rest[:NBUF]
    gsems = rest[NBUF:2 * NBUF]
    ssems = rest[2 * NBUF:3 * NBUF]

    wid = lax.axis_index("s") * NC + lax.axis_index("c")
    rbase = wid * (nch * CHUNK)

    pltpu.sync_copy(x_hbm.at[pl.ds(wid * nch, nch)], idx_v)
    pltpu.sync_copy(pos2_hbm, pos_v)

    def issue_gather(c, b):
        pltpu.async_copy(wt_hbm.at[idx_v.at[c]], bufs[b], gsems[b])

    def wait_gather(c, b):
        pltpu.make_async_copy(wt_hbm.at[idx_v.at[c]], bufs[b], gsems[b]).wait()

    def _store_refs(c, b):
        return (bufs[b],
                out_hbm.at[pl.ds(rbase + c * CHUNK, CHUNK), pl.ds(0, EMBED)])

    def issue_store(c, b):
        src, dst = _store_refs(c, b)
        pltpu.async_copy(src, dst, ssems[b])

    def wait_store(c, b):
        src, dst = _store_refs(c, b)
        pltpu.make_async_copy(src, dst, ssems[b]).wait()

    def add_pos(c, b):
        q = lax.rem(c * CHUNK, MAXLEN)
        buf = bufs[b]

        @plsc.parallel_loop(0, CHUNK, 1, unroll=4)
        def _(r):
            p = q + r
            for k in range(VPR):
                v = pos_v[p, pl.ds(k * LANES, LANES)]
                plsc.addupdate(buf.at[r, pl.ds(k * LANES, LANES)], v)

    def process(c, b):
        wait_gather(c, b)
        add_pos(c, b)
        issue_store(c, b)

    for b in range(LOOK):
        issue_gather(b, b)

    for c in range(LOOK):
        process(c, c % NBUF)
        issue_gather(c + LOOK, (c + LOOK) % NBUF)

    ngroups = (nch - 2 * LOOK) // NBUF

    def group(g, _):
        for j in range(NBUF):
            c = LOOK + g * NBUF + j
            b = (LOOK + j) % NBUF
            bf = (LOOK + j + LOOK) % NBUF
            wait_store(c - LOOK, bf)
            issue_gather(c + LOOK, bf)
            process(c, b)
        return _

    lax.fori_loop(0, ngroups, group, 0)

    for c in range(nch - LOOK, nch):
        process(c, c % NBUF)

    for c in range(nch - NBUF, nch):
        wait_store(c, c % NBUF)


def kernel(x, word_table, pos_table):
    batch, maxlen = x.shape
    rows = batch * maxlen
    assert maxlen == MAXLEN and word_table.shape[1] == EMBED
    rpw = rows // NW
    nch = rpw // CHUNK
    assert rpw % CHUNK == 0 and rpw % MAXLEN == 0
    assert (nch - 2 * LOOK) % NBUF == 0

    x2 = x.reshape(-1, CHUNK).astype(jnp.int32)
    wt2 = _tc_pack(word_table.T)
    pos2 = jnp.concatenate([pos_table, pos_table], axis=0)

    scratch = (
        [pltpu.VMEM((nch, CHUNK), jnp.int32),
         pltpu.VMEM((2 * MAXLEN, EMBED), jnp.float32)]
        + [pltpu.VMEM((CHUNK, EMBED), jnp.float32) for _ in range(NBUF)]
        + [pltpu.SemaphoreType.DMA for _ in range(2 * NBUF)]
    )
    out = pl.kernel(
        functools.partial(_emb_body, nch),
        out_type=jax.ShapeDtypeStruct((rows, WROW), jnp.float32),
        mesh=_mesh,
        scratch_types=scratch,
        compiler_params=pltpu.CompilerParams(use_tc_tiling_on_sc=False),
    )(x2, wt2, pos2)
    return out[:, :EMBED].reshape(batch, maxlen, EMBED)


# speedup vs baseline: 2.0191x; 1.2220x over previous
import functools

import jax
import jax.numpy as jnp
from jax import lax
from jax.experimental import pallas as pl
from jax.experimental.pallas import tpu as pltpu
from jax.experimental.pallas import tpu_sc as plsc

MAXLEN = 200
EMBED = 64
WROW = 128
LANES = 16
VPR = EMBED // LANES

NC, NS = 2, 16
NW = NC * NS

CHUNK = 128
NBUF = 4
LOOK = 2

_mesh = plsc.VectorSubcoreMesh(core_axis_name="c", subcore_axis_name="s")

VB = 8192
VBH = VB // 2


def _pack_body(in_ref, out_ref):
    t = in_ref[...]
    eye = jnp.eye(EMBED, dtype=jnp.float32)
    dn = (((0,), (0,)), ((), ()))
    left = jax.lax.dot_general(t[:, :VBH], eye, dn,
                               preferred_element_type=jnp.float32)
    right = jax.lax.dot_general(t[:, VBH:], eye, dn,
                                preferred_element_type=jnp.float32)
    out_ref[...] = jnp.concatenate([left, right], axis=1)


def _tc_pack(wt_t):
    v = wt_t.shape[1]
    grid = -(-v // VB)
    out = pl.pallas_call(
        _pack_body,
        grid=(grid,),
        in_specs=[pl.BlockSpec((EMBED, VB), lambda g: (0, g))],
        out_specs=pl.BlockSpec((VBH, 128), lambda g: (g, 0)),
        out_shape=jax.ShapeDtypeStruct((grid * VBH, 128), jnp.float32),
    )(wt_t)
    return out.reshape(grid * VB, EMBED)


def _emb_body(nch, x_hbm, wt_hbm, pos2_hbm, out_hbm, idx_v, pos_v, *rest):
    bufs = rest[:NBUF]
    gsems = rest[NBUF:2 * NBUF]
    ssems = rest[2 * NBUF:3 * NBUF]

    wid = lax.axis_index("s") * NC + lax.axis_index("c")
    rbase = wid * (nch * CHUNK)

    pltpu.sync_copy(x_hbm.at[pl.ds(wid * nch, nch)], idx_v)
    pltpu.sync_copy(pos2_hbm, pos_v)

    @plsc.parallel_loop(0, nch, 1, unroll=2)
    def _(r):
        for k in range(CHUNK // LANES):
            v = idx_v[r, pl.ds(k * LANES, LANES)]
            j = jnp.bitwise_and(v, VB - 1)
            half = jax.lax.shift_right_logical(j, 12)
            lo = jnp.bitwise_and(j, VBH - 1)
            idx_v[r, pl.ds(k * LANES, LANES)] = (v - j) + lo + lo + half

    def issue_gather(c, b):
        pltpu.async_copy(wt_hbm.at[idx_v.at[c]], bufs[b], gsems[b])

    def wait_gather(c, b):
        pltpu.make_async_copy(wt_hbm.at[idx_v.at[c]], bufs[b], gsems[b]).wait()

    def _store_refs(c, b):
        return (bufs[b],
                out_hbm.at[pl.ds(rbase + c * CHUNK, CHUNK), pl.ds(0, EMBED)])

    def issue_store(c, b):
        src, dst = _store_refs(c, b)
        pltpu.async_copy(src, dst, ssems[b])

    def wait_store(c, b):
        src, dst = _store_refs(c, b)
        pltpu.make_async_copy(src, dst, ssems[b]).wait()

    def add_pos(c, b):
        q = lax.rem(c * CHUNK, MAXLEN)
        buf = bufs[b]

        @plsc.parallel_loop(0, CHUNK, 1, unroll=4)
        def _(r):
            p = q + r
            for k in range(VPR):
                v = pos_v[p, pl.ds(k * LANES, LANES)]
                plsc.addupdate(buf.at[r, pl.ds(k * LANES, LANES)], v)

    def process(c, b):
        wait_gather(c, b)
        add_pos(c, b)
        issue_store(c, b)

    for b in range(LOOK):
        issue_gather(b, b)

    for c in range(LOOK):
        process(c, c % NBUF)
        issue_gather(c + LOOK, (c + LOOK) % NBUF)

    ngroups = (nch - 2 * LOOK) // NBUF

    def group(g, _):
        for j in range(NBUF):
            c = LOOK + g * NBUF + j
            b = (LOOK + j) % NBUF
            bf = (LOOK + j + LOOK) % NBUF
            wait_store(c - LOOK, bf)
            issue_gather(c + LOOK, bf)
            process(c, b)
        return _

    lax.fori_loop(0, ngroups, group, 0)

    for c in range(nch - LOOK, nch):
        process(c, c % NBUF)

    for c in range(nch - NBUF, nch):
        wait_store(c, c % NBUF)


def kernel(x, word_table, pos_table):
    batch, maxlen = x.shape
    rows = batch * maxlen
    assert maxlen == MAXLEN and word_table.shape[1] == EMBED
    rpw = rows // NW
    nch = rpw // CHUNK
    assert rpw % CHUNK == 0 and rpw % MAXLEN == 0
    assert (nch - 2 * LOOK) % NBUF == 0

    x2 = x.reshape(-1, CHUNK).astype(jnp.int32)
    wt2 = _tc_pack(word_table.T)
    pos2 = jnp.concatenate([pos_table, pos_table], axis=0)

    scratch = (
        [pltpu.VMEM((nch, CHUNK), jnp.int32),
         pltpu.VMEM((2 * MAXLEN, EMBED), jnp.float32)]
        + [pltpu.VMEM((CHUNK, EMBED), jnp.float32) for _ in range(NBUF)]
        + [pltpu.SemaphoreType.DMA for _ in range(2 * NBUF)]
    )
    out = pl.kernel(
        functools.partial(_emb_body, nch),
        out_type=jax.ShapeDtypeStruct((rows, WROW), jnp.float32),
        mesh=_mesh,
        scratch_types=scratch,
        compiler_params=pltpu.CompilerParams(use_tc_tiling_on_sc=False),
    )(x2, wt2, pos2)
    return out[:, :EMBED].reshape(batch, maxlen, EMBED)


# speedup vs baseline: 2.0225x; 1.0017x over previous
import functools

import jax
import jax.numpy as jnp
from jax import lax
from jax.experimental import pallas as pl
from jax.experimental.pallas import tpu as pltpu
from jax.experimental.pallas import tpu_sc as plsc

MAXLEN = 200
EMBED = 64
WROW = 128
LANES = 16
VPR = EMBED // LANES

NC, NS = 2, 16
NW = NC * NS

CHUNK = 128
NBUF = 4
LOOK = 2

_mesh = plsc.VectorSubcoreMesh(core_axis_name="c", subcore_axis_name="s")

VB = 8192
VBH = VB // 2


def _pack_body(in_ref, out_ref):
    t = in_ref[...]
    out_ref[...] = jnp.concatenate([t[:, :VBH].T, t[:, VBH:].T], axis=1)


def _tc_pack(wt_t):
    v = wt_t.shape[1]
    grid = -(-v // VB)
    out = pl.pallas_call(
        _pack_body,
        grid=(grid,),
        in_specs=[pl.BlockSpec((EMBED, VB), lambda g: (0, g))],
        out_specs=pl.BlockSpec((VBH, 128), lambda g: (g, 0)),
        out_shape=jax.ShapeDtypeStruct((grid * VBH, 128), jnp.float32),
    )(wt_t)
    return out.reshape(grid * VB, EMBED)


def _emb_body(nch, x_hbm, wt_hbm, pos2_hbm, out_hbm, idx_v, pos_v, *rest):
    bufs = rest[:NBUF]
    gsems = rest[NBUF:2 * NBUF]
    ssems = rest[2 * NBUF:3 * NBUF]

    wid = lax.axis_index("s") * NC + lax.axis_index("c")
    rbase = wid * (nch * CHUNK)

    pltpu.sync_copy(x_hbm.at[pl.ds(wid * nch, nch)], idx_v)
    pltpu.sync_copy(pos2_hbm, pos_v)

    @plsc.parallel_loop(0, nch, 1, unroll=2)
    def _(r):
        for k in range(CHUNK // LANES):
            v = idx_v[r, pl.ds(k * LANES, LANES)]
            j = jnp.bitwise_and(v, VB - 1)
            half = jax.lax.shift_right_logical(j, 12)
            lo = jnp.bitwise_and(j, VBH - 1)
            idx_v[r, pl.ds(k * LANES, LANES)] = (v - j) + lo + lo + half

    def issue_gather(c, b):
        pltpu.async_copy(wt_hbm.at[idx_v.at[c]], bufs[b], gsems[b])

    def wait_gather(c, b):
        pltpu.make_async_copy(wt_hbm.at[idx_v.at[c]], bufs[b], gsems[b]).wait()

    def _store_refs(c, b):
        return (bufs[b],
                out_hbm.at[pl.ds(rbase + c * CHUNK, CHUNK), pl.ds(0, EMBED)])

    def issue_store(c, b):
        src, dst = _store_refs(c, b)
        pltpu.async_copy(src, dst, ssems[b])

    def wait_store(c, b):
        src, dst = _store_refs(c, b)
        pltpu.make_async_copy(src, dst, ssems[b]).wait()

    def add_pos(c, b):
        q = lax.rem(c * CHUNK, MAXLEN)
        buf = bufs[b]

        @plsc.parallel_loop(0, CHUNK, 1, unroll=4)
        def _(r):
            p = q + r
            for k in range(VPR):
                v = pos_v[p, pl.ds(k * LANES, LANES)]
                plsc.addupdate(buf.at[r, pl.ds(k * LANES, LANES)], v)

    def process(c, b):
        wait_gather(c, b)
        add_pos(c, b)
        issue_store(c, b)

    for b in range(LOOK):
        issue_gather(b, b)

    for c in range(LOOK):
        process(c, c % NBUF)
        issue_gather(c + LOOK, (c + LOOK) % NBUF)

    ngroups = (nch - 2 * LOOK) // NBUF

    def group(g, _):
        for j in range(NBUF):
            c = LOOK + g * NBUF + j
            b = (LOOK + j) % NBUF
            bf = (LOOK + j + LOOK) % NBUF
            wait_store(c - LOOK, bf)
            issue_gather(c + LOOK, bf)
            process(c, b)
        return _

    lax.fori_loop(0, ngroups, group, 0)

    for c in range(nch - LOOK, nch):
        process(c, c % NBUF)

    for c in range(nch - NBUF, nch):
        wait_store(c, c % NBUF)


def kernel(x, word_table, pos_table):
    batch, maxlen = x.shape
    rows = batch * maxlen
    assert maxlen == MAXLEN and word_table.shape[1] == EMBED
    rpw = rows // NW
    nch = rpw // CHUNK
    assert rpw % CHUNK == 0 and rpw % MAXLEN == 0
    assert (nch - 2 * LOOK) % NBUF == 0

    x2 = x.reshape(-1, CHUNK).astype(jnp.int32)
    wt2 = _tc_pack(word_table.T)
    pos2 = jnp.concatenate([pos_table, pos_table], axis=0)

    scratch = (
        [pltpu.VMEM((nch, CHUNK), jnp.int32),
         pltpu.VMEM((2 * MAXLEN, EMBED), jnp.float32)]
        + [pltpu.VMEM((CHUNK, EMBED), jnp.float32) for _ in range(NBUF)]
        + [pltpu.SemaphoreType.DMA for _ in range(2 * NBUF)]
    )
    out = pl.kernel(
        functools.partial(_emb_body, nch),
        out_type=jax.ShapeDtypeStruct((rows, WROW), jnp.float32),
        mesh=_mesh,
        scratch_types=scratch,
        compiler_params=pltpu.CompilerParams(use_tc_tiling_on_sc=False),
    )(x2, wt2, pos2)
    return out[:, :EMBED].reshape(batch, maxlen, EMBED)


# speedup vs baseline: 2.0386x; 1.0080x over previous
import functools

import jax
import jax.numpy as jnp
from jax import lax
from jax.experimental import pallas as pl
from jax.experimental.pallas import tpu as pltpu
from jax.experimental.pallas import tpu_sc as plsc

MAXLEN = 200
EMBED = 64
WROW = 128
LANES = 16
VPR = EMBED // LANES

NC, NS = 2, 16
NW = NC * NS

CHUNK = 128
NBUF = 8
LOOK = 4

_mesh = plsc.VectorSubcoreMesh(core_axis_name="c", subcore_axis_name="s")

VB = 8192
VBH = VB // 2


def _pack_body(in_ref, out_ref):
    t = in_ref[...]
    out_ref[...] = jnp.concatenate([t[:, :VBH].T, t[:, VBH:].T], axis=1)


def _tc_pack(wt_t):
    v = wt_t.shape[1]
    grid = -(-v // VB)
    out = pl.pallas_call(
        _pack_body,
        grid=(grid,),
        in_specs=[pl.BlockSpec((EMBED, VB), lambda g: (0, g))],
        out_specs=pl.BlockSpec((VBH, 128), lambda g: (g, 0)),
        out_shape=jax.ShapeDtypeStruct((grid * VBH, 128), jnp.float32),
    )(wt_t)
    return out.reshape(grid * VB, EMBED)


def _emb_body(nch, x_hbm, wt_hbm, pos2_hbm, out_hbm, idx_v, pos_v, *rest):
    bufs = rest[:NBUF]
    gsems = rest[NBUF:2 * NBUF]
    ssems = rest[2 * NBUF:3 * NBUF]

    wid = lax.axis_index("s") * NC + lax.axis_index("c")
    rbase = wid * (nch * CHUNK)

    pltpu.sync_copy(x_hbm.at[pl.ds(wid * nch, nch)], idx_v)
    pltpu.sync_copy(pos2_hbm, pos_v)

    @plsc.parallel_loop(0, nch, 1, unroll=2)
    def _(r):
        for k in range(CHUNK // LANES):
            v = idx_v[r, pl.ds(k * LANES, LANES)]
            j = jnp.bitwise_and(v, VB - 1)
            half = jax.lax.shift_right_logical(j, 12)
            lo = jnp.bitwise_and(j, VBH - 1)
            idx_v[r, pl.ds(k * LANES, LANES)] = (v - j) + lo + lo + half

    def issue_gather(c, b):
        pltpu.async_copy(wt_hbm.at[idx_v.at[c]], bufs[b], gsems[b])

    def wait_gather(c, b):
        pltpu.make_async_copy(wt_hbm.at[idx_v.at[c]], bufs[b], gsems[b]).wait()

    def _store_refs(c, b):
        return (bufs[b],
                out_hbm.at[pl.ds(rbase + c * CHUNK, CHUNK), pl.ds(0, EMBED)])

    def issue_store(c, b):
        src, dst = _store_refs(c, b)
        pltpu.async_copy(src, dst, ssems[b])

    def wait_store(c, b):
        src, dst = _store_refs(c, b)
        pltpu.make_async_copy(src, dst, ssems[b]).wait()

    def add_pos(c, b):
        q = lax.rem(c * CHUNK, MAXLEN)
        buf = bufs[b]

        @plsc.parallel_loop(0, CHUNK, 1, unroll=4)
        def _(r):
            p = q + r
            for k in range(VPR):
                v = pos_v[p, pl.ds(k * LANES, LANES)]
                plsc.addupdate(buf.at[r, pl.ds(k * LANES, LANES)], v)

    def process(c, b):
        wait_gather(c, b)
        add_pos(c, b)
        issue_store(c, b)

    for b in range(LOOK):
        issue_gather(b, b)

    for c in range(LOOK):
        process(c, c % NBUF)
        issue_gather(c + LOOK, (c + LOOK) % NBUF)

    ngroups = (nch - 2 * LOOK) // NBUF

    def group(g, _):
        for j in range(NBUF):
            c = LOOK + g * NBUF + j
            b = (LOOK + j) % NBUF
            bf = (LOOK + j + LOOK) % NBUF
            wait_store(c - LOOK, bf)
            issue_gather(c + LOOK, bf)
            process(c, b)
        return _

    lax.fori_loop(0, ngroups, group, 0)

    for c in range(nch - LOOK, nch):
        process(c, c % NBUF)

    for c in range(nch - NBUF, nch):
        wait_store(c, c % NBUF)


def kernel(x, word_table, pos_table):
    batch, maxlen = x.shape
    rows = batch * maxlen
    assert maxlen == MAXLEN and word_table.shape[1] == EMBED
    rpw = rows // NW
    nch = rpw // CHUNK
    assert rpw % CHUNK == 0 and rpw % MAXLEN == 0
    assert (nch - 2 * LOOK) % NBUF == 0

    x2 = x.reshape(-1, CHUNK).astype(jnp.int32)
    wt2 = _tc_pack(word_table.T)
    pos2 = jnp.concatenate([pos_table, pos_table], axis=0)

    scratch = (
        [pltpu.VMEM((nch, CHUNK), jnp.int32),
         pltpu.VMEM((2 * MAXLEN, EMBED), jnp.float32)]
        + [pltpu.VMEM((CHUNK, EMBED), jnp.float32) for _ in range(NBUF)]
        + [pltpu.SemaphoreType.DMA for _ in range(2 * NBUF)]
    )
    out = pl.kernel(
        functools.partial(_emb_body, nch),
        out_type=jax.ShapeDtypeStruct((rows, WROW), jnp.float32),
        mesh=_mesh,
        scratch_types=scratch,
        compiler_params=pltpu.CompilerParams(use_tc_tiling_on_sc=False),
    )(x2, wt2, pos2)
    return out[:, :EMBED].reshape(batch, maxlen, EMBED)


# speedup vs baseline: 2.1493x; 1.0543x over previous
import functools

import jax
import jax.numpy as jnp
from jax import lax
from jax.experimental import pallas as pl
from jax.experimental.pallas import tpu as pltpu
from jax.experimental.pallas import tpu_sc as plsc

MAXLEN = 200
EMBED = 64
WROW = 128
LANES = 16
VPR = EMBED // LANES

NC, NS = 2, 16
NW = NC * NS

CHUNK = 128
NBUF = 8
LOOK = 4

_mesh = plsc.VectorSubcoreMesh(core_axis_name="c", subcore_axis_name="s")

VB = 16384
VBH = VB // 2
VSHIFT = VBH.bit_length() - 1


def _pack_body(in_ref, out_ref):
    t = in_ref[...]
    out_ref[...] = jnp.concatenate([t[:, :VBH].T, t[:, VBH:].T], axis=1)


def _tc_pack(wt_t):
    v = wt_t.shape[1]
    grid = -(-v // VB)
    out = pl.pallas_call(
        _pack_body,
        grid=(grid,),
        in_specs=[pl.BlockSpec((EMBED, VB), lambda g: (0, g))],
        out_specs=pl.BlockSpec((VBH, 128), lambda g: (g, 0)),
        out_shape=jax.ShapeDtypeStruct((grid * VBH, 128), jnp.float32),
    )(wt_t)
    return out.reshape(grid * VB, EMBED)


def _emb_body(nch, x_hbm, wt_hbm, pos2_hbm, out_hbm, idx_v, pos_v, *rest):
    bufs = rest[:NBUF]
    gsems = rest[NBUF:2 * NBUF]
    ssems = rest[2 * NBUF:3 * NBUF]

    wid = lax.axis_index("s") * NC + lax.axis_index("c")
    rbase = wid * (nch * CHUNK)

    pltpu.sync_copy(x_hbm.at[pl.ds(wid * nch, nch)], idx_v)
    pltpu.sync_copy(pos2_hbm, pos_v)

    @plsc.parallel_loop(0, nch, 1, unroll=2)
    def _(r):
        for k in range(CHUNK // LANES):
            v = idx_v[r, pl.ds(k * LANES, LANES)]
            j = jnp.bitwise_and(v, VB - 1)
            half = jax.lax.shift_right_logical(j, VSHIFT)
            lo = jnp.bitwise_and(j, VBH - 1)
            idx_v[r, pl.ds(k * LANES, LANES)] = (v - j) + lo + lo + half

    def issue_gather(c, b):
        pltpu.async_copy(wt_hbm.at[idx_v.at[c]], bufs[b], gsems[b])

    def wait_gather(c, b):
        pltpu.make_async_copy(wt_hbm.at[idx_v.at[c]], bufs[b], gsems[b]).wait()

    def _store_refs(c, b):
        return (bufs[b],
                out_hbm.at[pl.ds(rbase + c * CHUNK, CHUNK), pl.ds(0, EMBED)])

    def issue_store(c, b):
        src, dst = _store_refs(c, b)
        pltpu.async_copy(src, dst, ssems[b])

    def wait_store(c, b):
        src, dst = _store_refs(c, b)
        pltpu.make_async_copy(src, dst, ssems[b]).wait()

    def add_pos(c, b):
        q = lax.rem(c * CHUNK, MAXLEN)
        buf = bufs[b]

        @plsc.parallel_loop(0, CHUNK, 1, unroll=4)
        def _(r):
            p = q + r
            for k in range(VPR):
                v = pos_v[p, pl.ds(k * LANES, LANES)]
                plsc.addupdate(buf.at[r, pl.ds(k * LANES, LANES)], v)

    def process(c, b):
        wait_gather(c, b)
        add_pos(c, b)
        issue_store(c, b)

    for b in range(LOOK):
        issue_gather(b, b)

    for c in range(LOOK):
        process(c, c % NBUF)
        issue_gather(c + LOOK, (c + LOOK) % NBUF)

    ngroups = (nch - 2 * LOOK) // NBUF

    def group(g, _):
        for j in range(NBUF):
            c = LOOK + g * NBUF + j
            b = (LOOK + j) % NBUF
            bf = (LOOK + j + LOOK) % NBUF
            wait_store(c - LOOK, bf)
            issue_gather(c + LOOK, bf)
            process(c, b)
        return _

    lax.fori_loop(0, ngroups, group, 0)

    for c in range(nch - LOOK, nch):
        process(c, c % NBUF)

    for c in range(nch - NBUF, nch):
        wait_store(c, c % NBUF)


def kernel(x, word_table, pos_table):
    batch, maxlen = x.shape
    rows = batch * maxlen
    assert maxlen == MAXLEN and word_table.shape[1] == EMBED
    rpw = rows // NW
    nch = rpw // CHUNK
    assert rpw % CHUNK == 0 and rpw % MAXLEN == 0
    assert (nch - 2 * LOOK) % NBUF == 0

    x2 = x.reshape(-1, CHUNK).astype(jnp.int32)
    wt2 = _tc_pack(word_table.T)
    pos2 = jnp.concatenate([pos_table, pos_table], axis=0)

    scratch = (
        [pltpu.VMEM((nch, CHUNK), jnp.int32),
         pltpu.VMEM((2 * MAXLEN, EMBED), jnp.float32)]
        + [pltpu.VMEM((CHUNK, EMBED), jnp.float32) for _ in range(NBUF)]
        + [pltpu.SemaphoreType.DMA for _ in range(2 * NBUF)]
    )
    out = pl.kernel(
        functools.partial(_emb_body, nch),
        out_type=jax.ShapeDtypeStruct((rows, WROW), jnp.float32),
        mesh=_mesh,
        scratch_types=scratch,
        compiler_params=pltpu.CompilerParams(use_tc_tiling_on_sc=False),
    )(x2, wt2, pos2)
    return out[:, :EMBED].reshape(batch, maxlen, EMBED)


# speedup vs baseline: 2.2074x; 1.0270x over previous
import functools

import jax
import jax.numpy as jnp
from jax import lax
from jax.experimental import pallas as pl
from jax.experimental.pallas import tpu as pltpu
from jax.experimental.pallas import tpu_sc as plsc

MAXLEN = 200
EMBED = 64
WROW = 128
LANES = 16
VPR = EMBED // LANES

NC, NS = 2, 16
NW = NC * NS

CHUNK = 128
NBUF = 8
LOOK = 4

_mesh = plsc.VectorSubcoreMesh(core_axis_name="c", subcore_axis_name="s")

VB = 32768
VBH = VB // 2
VSHIFT = VBH.bit_length() - 1


def _pack_body(in_ref, out_ref):
    t = in_ref[...]
    out_ref[...] = jnp.concatenate([t[:, :VBH].T, t[:, VBH:].T], axis=1)


def _tc_pack(wt_t):
    v = wt_t.shape[1]
    grid = -(-v // VB)
    out = pl.pallas_call(
        _pack_body,
        grid=(grid,),
        in_specs=[pl.BlockSpec((EMBED, VB), lambda g: (0, g))],
        out_specs=pl.BlockSpec((VBH, 128), lambda g: (g, 0)),
        out_shape=jax.ShapeDtypeStruct((grid * VBH, 128), jnp.float32),
    )(wt_t)
    return out.reshape(grid * VB, EMBED)


def _emb_body(nch, x_hbm, wt_hbm, pos2_hbm, out_hbm, idx_v, pos_v, *rest):
    bufs = rest[:NBUF]
    gsems = rest[NBUF:2 * NBUF]
    ssems = rest[2 * NBUF:3 * NBUF]

    wid = lax.axis_index("s") * NC + lax.axis_index("c")
    rbase = wid * (nch * CHUNK)

    pltpu.sync_copy(x_hbm.at[pl.ds(wid * nch, nch)], idx_v)
    pltpu.sync_copy(pos2_hbm, pos_v)

    @plsc.parallel_loop(0, nch, 1, unroll=2)
    def _(r):
        for k in range(CHUNK // LANES):
            v = idx_v[r, pl.ds(k * LANES, LANES)]
            j = jnp.bitwise_and(v, VB - 1)
            half = jax.lax.shift_right_logical(j, VSHIFT)
            lo = jnp.bitwise_and(j, VBH - 1)
            idx_v[r, pl.ds(k * LANES, LANES)] = (v - j) + lo + lo + half

    def issue_gather(c, b):
        pltpu.async_copy(wt_hbm.at[idx_v.at[c]], bufs[b], gsems[b])

    def wait_gather(c, b):
        pltpu.make_async_copy(wt_hbm.at[idx_v.at[c]], bufs[b], gsems[b]).wait()

    def _store_refs(c, b):
        return (bufs[b],
                out_hbm.at[pl.ds(rbase + c * CHUNK, CHUNK), pl.ds(0, EMBED)])

    def issue_store(c, b):
        src, dst = _store_refs(c, b)
        pltpu.async_copy(src, dst, ssems[b])

    def wait_store(c, b):
        src, dst = _store_refs(c, b)
        pltpu.make_async_copy(src, dst, ssems[b]).wait()

    def add_pos(c, b):
        q = lax.rem(c * CHUNK, MAXLEN)
        buf = bufs[b]

        @plsc.parallel_loop(0, CHUNK, 1, unroll=4)
        def _(r):
            p = q + r
            for k in range(VPR):
                v = pos_v[p, pl.ds(k * LANES, LANES)]
                plsc.addupdate(buf.at[r, pl.ds(k * LANES, LANES)], v)

    def process(c, b):
        wait_gather(c, b)
        add_pos(c, b)
        issue_store(c, b)

    for b in range(LOOK):
        issue_gather(b, b)

    for c in range(LOOK):
        process(c, c % NBUF)
        issue_gather(c + LOOK, (c + LOOK) % NBUF)

    ngroups = (nch - 2 * LOOK) // NBUF

    def group(g, _):
        for j in range(NBUF):
            c = LOOK + g * NBUF + j
            b = (LOOK + j) % NBUF
            bf = (LOOK + j + LOOK) % NBUF
            wait_store(c - LOOK, bf)
            issue_gather(c + LOOK, bf)
            process(c, b)
        return _

    lax.fori_loop(0, ngroups, group, 0)

    for c in range(nch - LOOK, nch):
        process(c, c % NBUF)

    for c in range(nch - NBUF, nch):
        wait_store(c, c % NBUF)


def kernel(x, word_table, pos_table):
    batch, maxlen = x.shape
    rows = batch * maxlen
    assert maxlen == MAXLEN and word_table.shape[1] == EMBED
    rpw = rows // NW
    nch = rpw // CHUNK
    assert rpw % CHUNK == 0 and rpw % MAXLEN == 0
    assert (nch - 2 * LOOK) % NBUF == 0

    x2 = x.reshape(-1, CHUNK).astype(jnp.int32)
    wt2 = _tc_pack(word_table.T)
    pos2 = jnp.concatenate([pos_table, pos_table], axis=0)

    scratch = (
        [pltpu.VMEM((nch, CHUNK), jnp.int32),
         pltpu.VMEM((2 * MAXLEN, EMBED), jnp.float32)]
        + [pltpu.VMEM((CHUNK, EMBED), jnp.float32) for _ in range(NBUF)]
        + [pltpu.SemaphoreType.DMA for _ in range(2 * NBUF)]
    )
    out = pl.kernel(
        functools.partial(_emb_body, nch),
        out_type=jax.ShapeDtypeStruct((rows, WROW), jnp.float32),
        mesh=_mesh,
        scratch_types=scratch,
        compiler_params=pltpu.CompilerParams(use_tc_tiling_on_sc=False),
    )(x2, wt2, pos2)
    return out[:, :EMBED].reshape(batch, maxlen, EMBED)
